# baseline (speedup 1.0000x reference)
import jax
import jax.numpy as jnp
from jax import lax
from jax.experimental import pallas as pl
from jax.experimental.pallas import tpu as pltpu
from jax.experimental.pallas import tpu_sc as plsc

NC = 2
NS = 16
NW = NC * NS
L = 16
CHUNK = 128
G = 8


def _matmul_body(x_ref, wd_ref, w2_ref, y1_ref, y2_ref):
    x = x_ref[...]
    dn = (((0,), (0,)), ((), ()))
    y1_ref[...] = lax.dot_general(x, wd_ref[...], dn,
                                  preferred_element_type=jnp.float32)
    y2_ref[...] = lax.dot_general(x, w2_ref[...], dn,
                                  preferred_element_type=jnp.float32)


def _stage1(X, Wd_t, W2_t):
    C, N = X.shape
    O = Wd_t.shape[1]
    return pl.pallas_call(
        _matmul_body,
        out_shape=[
            jax.ShapeDtypeStruct((N, O), jnp.float32),
            jax.ShapeDtypeStruct((N, O), jnp.float32),
        ],
    )(X, Wd_t, W2_t)


def _make_sc_kernel(N, O, NPAD, CPW):
    assert CPW % (2 * G) == 0
    pair_iters = CPW // (2 * G)
    row_grps = N // CHUNK
    row_tail = NPAD - row_grps * CHUNK
    pub_tail = N - row_grps * CHUNK
    assert row_tail % 8 == 0 and pub_tail % 8 == 0

    mesh = plsc.VectorSubcoreMesh(core_axis_name="c", subcore_axis_name="s")

    def body(y2_hbm, ridx_hbm, gidx_hbm, z128_hbm, zhist_hbm,
             s_out, cnt_out,
             s_sh, idxg0, idxg1, idxr0, idxr1, rows0, rows1,
             hist_v, sem0, sem1):
        cid = lax.axis_index("c")
        sid = lax.axis_index("s")
        wid = sid * NC + cid
        c0 = wid * CPW

        idxg = (idxg0, idxg1)
        idxr = (idxr0, idxr1)
        rows = (rows0, rows1)
        sems = (sem0, sem1)

        pltpu.sync_copy(z128_hbm, rows0)
        pltpu.sync_copy(zhist_hbm, hist_v)

        def zero_grp(g, carry):
            o = (sid + g * NS) * CHUNK
            pltpu.sync_copy(rows0, s_sh.at[pl.ds(o, CHUNK)])
            return carry

        lax.fori_loop(0, row_grps // NS, zero_grp, 0)
        rem = row_grps % NS

        @pl.when(sid < rem)
        def _zero_rem():
            o = ((row_grps // NS) * NS + sid) * CHUNK
            pltpu.sync_copy(rows0, s_sh.at[pl.ds(o, CHUNK)])

        if row_tail:
            @pl.when(sid == NS - 1)
            def _zero_tail():
                t0 = row_grps * CHUNK
                pltpu.sync_copy(rows0.at[pl.ds(0, row_tail)],
                                s_sh.at[pl.ds(t0, row_tail)])
        plsc.subcore_barrier()

        def load_group(t, gi, slot):
            base = c0 + t * 2 * G + gi * G
            pltpu.sync_copy(gidx_hbm.at[pl.ds(base, G)], idxg[slot])
            pltpu.sync_copy(ridx_hbm.at[pl.ds(base, G)], idxr[slot])

        def stage(p):
            slot, row = (0, p) if p < G else (1, p - G)
            buf = p % 2
            pltpu.make_async_copy(y2_hbm.at[idxg[slot].at[row]],
                                  rows[buf], sems[buf]).start()

        def hist(p):
            slot, row = (0, p) if p < G else (1, p - G)
            for u in range(CHUNK // L):
                iv = idxr[slot][row, pl.ds(u * L, L)]
                cnts, last = plsc.scan_count(iv)
                plsc.addupdate_scatter(hist_v, [iv],
                                       cnts.astype(jnp.float32), mask=last)

        def finish(p):
            slot, row = (0, p) if 0 <= p < G else (1, p % (2 * G) - G)
            buf = p % 2
            pltpu.make_async_copy(y2_hbm.at[idxg[slot].at[row]],
                                  rows[buf], sems[buf]).wait()
            pltpu.sync_copy(rows[buf], s_sh.at[idxr[slot].at[row]], add=True)

        def pair_body(t, first):
            del first
            for p in range(2 * G):
                if p == 0:
                    load_group(t, 0, 0)
                if p == G:
                    load_group(t, 1, 1)
                stage(p)
                hist(p)
                finish(p)

        pair_body(0, True)

        def loop_body(t, carry):
            pair_body(t, False)
            return carry

        lax.fori_loop(1, pair_iters, loop_body, 0)
        plsc.subcore_barrier()

        def pub_grp(g, carry):
            o = (sid + g * NS) * CHUNK
            pltpu.sync_copy(s_sh.at[pl.ds(o, CHUNK)], rows0)
            pltpu.sync_copy(rows0, s_out.at[cid, pl.ds(o, CHUNK)])
            return carry

        lax.fori_loop(0, row_grps // NS, pub_grp, 0)

        @pl.when(sid < rem)
        def _pub_rem():
            o = ((row_grps // NS) * NS + sid) * CHUNK
            pltpu.sync_copy(s_sh.at[pl.ds(o, CHUNK)], rows0)
            pltpu.sync_copy(rows0, s_out.at[cid, pl.ds(o, CHUNK)])

        if pub_tail:
            @pl.when(sid == NS - 1)
            def _pub_tail():
                t0 = row_grps * CHUNK
                pltpu.sync_copy(s_sh.at[pl.ds(t0, pub_tail)],
                                rows0.at[pl.ds(0, pub_tail)])
                pltpu.sync_copy(rows0.at[pl.ds(0, pub_tail)],
                                s_out.at[cid, pl.ds(t0, pub_tail)])

        pltpu.sync_copy(hist_v, cnt_out.at[cid, sid])

    return pl.kernel(
        body,
        out_type=[
            jax.ShapeDtypeStruct((NC, N, O), jnp.float32),
            jax.ShapeDtypeStruct((NC, NS, NPAD), jnp.float32),
        ],
        mesh=mesh,
        compiler_params=pltpu.CompilerParams(needs_layout_passes=False),
        scratch_types=[
            pltpu.VMEM_SHARED((NPAD, O), jnp.float32),
            pltpu.VMEM((G, CHUNK), jnp.int32),
            pltpu.VMEM((G, CHUNK), jnp.int32),
            pltpu.VMEM((G, CHUNK), jnp.int32),
            pltpu.VMEM((G, CHUNK), jnp.int32),
            pltpu.VMEM((CHUNK, O), jnp.float32),
            pltpu.VMEM((CHUNK, O), jnp.float32),
            pltpu.VMEM((NPAD,), jnp.float32),
            pltpu.SemaphoreType.DMA,
            pltpu.SemaphoreType.DMA,
        ],
    )


def _combine_body(y1_ref, s_ref, cnt_ref, b_ref, pw_ref, out_ref):
    s = s_ref[0] + s_ref[1]
    c = jnp.sum(cnt_ref[...], axis=0)[:, None]
    y = y1_ref[...] + b_ref[...]
    tot = c * y + s
    out = tot / jnp.maximum(c, 1.0)
    pw = pw_ref[0, 0]
    out = jnp.where(out >= 0, out, pw * out)
    out_ref[...] = out.T


def _stage3(Y1t, S, CNT, b2, pw2):
    N, O = Y1t.shape
    return pl.pallas_call(
        _combine_body,
        out_shape=jax.ShapeDtypeStruct((O, N), jnp.float32),
    )(Y1t, S, CNT, b2, pw2)


def kernel(in_features, reduce_index, gather_index, W, b, prelu_w):
    X = in_features[0]
    C, N = X.shape
    O = W.shape[0]
    E = reduce_index.shape[0]
    NPAD = N + 8
    span = NW * 2 * G * CHUNK
    EPAD = ((E + span - 1) // span) * span
    CPW = EPAD // (NW * CHUNK)

    ridx = reduce_index.astype(jnp.int32)
    gidx = gather_index.astype(jnp.int32)
    rpad = jnp.concatenate(
        [ridx, jnp.full((EPAD - E,), N, jnp.int32)]).reshape(-1, CHUNK)
    gpad = jnp.concatenate(
        [gidx, jnp.zeros((EPAD - E,), jnp.int32)]).reshape(-1, CHUNK)

    W1 = W[:, :C]
    W2 = W[:, C:]
    Wd_t = (W1 - W2).T
    W2_t = W2.T

    Y1t, Y2t = _stage1(X, Wd_t, W2_t)

    z128 = jnp.zeros((CHUNK, O), jnp.float32)
    zhist = jnp.zeros((NPAD,), jnp.float32)
    S, CNT = _make_sc_kernel(N, O, NPAD, CPW)(Y2t, rpad, gpad, z128, zhist)

    out2d = _stage3(Y1t, S, CNT[:, :, :N].reshape(NC * NS, N),
                    b.reshape(1, O), prelu_w.reshape(1, 1))
    return out2d[None]


# speedup vs baseline: 2.2955x; 2.2955x over previous
import jax
import jax.numpy as jnp
from jax import lax
from jax.experimental import pallas as pl
from jax.experimental.pallas import tpu as pltpu
from jax.experimental.pallas import tpu_sc as plsc

NC = 2
NS = 16
NW = NC * NS
L = 16
CHUNK = 128


def _matmul_body(x_ref, wd_ref, w2_ref, y1_ref, y2_ref):
    x = x_ref[...]
    dn = (((0,), (0,)), ((), ()))
    y1_ref[...] = lax.dot_general(x, wd_ref[...], dn,
                                  preferred_element_type=jnp.float32)
    y2_ref[...] = lax.dot_general(x, w2_ref[...], dn,
                                  preferred_element_type=jnp.float32)


def _stage1(X, Wd_t, W2_t):
    C, N = X.shape
    O = Wd_t.shape[1]
    return pl.pallas_call(
        _matmul_body,
        out_shape=[
            jax.ShapeDtypeStruct((N, O), jnp.float32),
            jax.ShapeDtypeStruct((N, O), jnp.float32),
        ],
    )(X, Wd_t, W2_t)


def _make_sc_kernel(N, O, E):
    assert E % CHUNK == 0
    tot_chunks = E // CHUNK
    base_chunks = tot_chunks // NW
    extra = tot_chunks % NW
    row_grps = N // CHUNK
    row_tail = N - row_grps * CHUNK
    assert row_tail % 8 == 0

    mesh = plsc.VectorSubcoreMesh(core_axis_name="c", subcore_axis_name="s")

    def body(y2_hbm, ridx_hbm, gidx_hbm, z128_hbm, zhist_hbm,
             s_out, cnt_out,
             s_sh, idxg_v, idxr_v, rows_v, hist_v, sem):
        cid = lax.axis_index("c")
        sid = lax.axis_index("s")
        wid = sid * NC + cid

        pltpu.sync_copy(z128_hbm, rows_v)
        pltpu.sync_copy(zhist_hbm, hist_v)

        def zero_grp(g, carry):
            o = (sid + g * NS) * CHUNK
            pltpu.sync_copy(rows_v, s_sh.at[pl.ds(o, CHUNK)])
            return carry

        lax.fori_loop(0, row_grps // NS, zero_grp, 0)
        rem = row_grps % NS

        @pl.when(sid < rem)
        def _zero_rem():
            o = ((row_grps // NS) * NS + sid) * CHUNK
            pltpu.sync_copy(rows_v, s_sh.at[pl.ds(o, CHUNK)])

        if row_tail:
            @pl.when(sid == NS - 1)
            def _zero_tail():
                t0 = row_grps * CHUNK
                pltpu.sync_copy(rows_v.at[pl.ds(0, row_tail)],
                                s_sh.at[pl.ds(t0, row_tail)])
        plsc.subcore_barrier()

        def chunk_c(j, carry):
            base = (wid * base_chunks + j) * CHUNK
            pltpu.sync_copy(gidx_hbm.at[pl.ds(base, CHUNK)], idxg_v)
            pltpu.sync_copy(ridx_hbm.at[pl.ds(base, CHUNK)], idxr_v)
            desc = pltpu.make_async_copy(y2_hbm.at[idxg_v], rows_v, sem)
            desc.start()
            for u in range(CHUNK // L):
                iv = idxr_v[pl.ds(u * L, L)]
                cnts, last = plsc.scan_count(iv)
                plsc.addupdate_scatter(hist_v, [iv],
                                       cnts.astype(jnp.float32), mask=last)
            desc.wait()
            pltpu.sync_copy(rows_v, s_sh.at[idxr_v], add=True)
            return carry

        lax.fori_loop(0, base_chunks, chunk_c, 0)
        if extra:
            @pl.when(wid < extra)
            def _extra():
                j = NW * base_chunks + wid
                base = j * CHUNK
                pltpu.sync_copy(gidx_hbm.at[pl.ds(base, CHUNK)], idxg_v)
                pltpu.sync_copy(ridx_hbm.at[pl.ds(base, CHUNK)], idxr_v)
                desc = pltpu.make_async_copy(y2_hbm.at[idxg_v], rows_v, sem)
                desc.start()
                for u in range(CHUNK // L):
                    iv = idxr_v[pl.ds(u * L, L)]
                    cnts, last = plsc.scan_count(iv)
                    plsc.addupdate_scatter(hist_v, [iv],
                                           cnts.astype(jnp.float32),
                                           mask=last)
                desc.wait()
                pltpu.sync_copy(rows_v, s_sh.at[idxr_v], add=True)
        plsc.subcore_barrier()

        def pub_grp(g, carry):
            o = (sid + g * NS) * CHUNK
            pltpu.sync_copy(s_sh.at[pl.ds(o, CHUNK)], rows_v)
            pltpu.sync_copy(rows_v, s_out.at[cid, pl.ds(o, CHUNK)])
            return carry

        lax.fori_loop(0, row_grps // NS, pub_grp, 0)

        @pl.when(sid < rem)
        def _pub_rem():
            o = ((row_grps // NS) * NS + sid) * CHUNK
            pltpu.sync_copy(s_sh.at[pl.ds(o, CHUNK)], rows_v)
            pltpu.sync_copy(rows_v, s_out.at[cid, pl.ds(o, CHUNK)])

        if row_tail:
            @pl.when(sid == NS - 1)
            def _pub_tail():
                t0 = row_grps * CHUNK
                pltpu.sync_copy(s_sh.at[pl.ds(t0, row_tail)],
                                rows_v.at[pl.ds(0, row_tail)])
                pltpu.sync_copy(rows_v.at[pl.ds(0, row_tail)],
                                s_out.at[cid, pl.ds(t0, row_tail)])

        pltpu.sync_copy(hist_v, cnt_out.at[cid, sid])

    return pl.kernel(
        body,
        out_type=[
            jax.ShapeDtypeStruct((NC, N, O), jnp.float32),
            jax.ShapeDtypeStruct((NC, NS, N), jnp.float32),
        ],
        mesh=mesh,
        compiler_params=pltpu.CompilerParams(needs_layout_passes=False),
        scratch_types=[
            pltpu.VMEM_SHARED((N, O), jnp.float32),
            pltpu.VMEM((CHUNK,), jnp.int32),
            pltpu.VMEM((CHUNK,), jnp.int32),
            pltpu.VMEM((CHUNK, O), jnp.float32),
            pltpu.VMEM((N,), jnp.float32),
            pltpu.SemaphoreType.DMA,
        ],
    )


def _combine_body(y1_ref, s_ref, cnt_ref, b_ref, pw_ref, out_ref):
    s = s_ref[0] + s_ref[1]
    c = jnp.sum(cnt_ref[...], axis=0)[:, None]
    y = y1_ref[...] + b_ref[...]
    tot = c * y + s
    out = tot / jnp.maximum(c, 1.0)
    pw = pw_ref[0, 0]
    out = jnp.where(out >= 0, out, pw * out)
    out_ref[...] = out.T


def _stage3(Y1t, S, CNT, b2, pw2):
    N, O = Y1t.shape
    return pl.pallas_call(
        _combine_body,
        out_shape=jax.ShapeDtypeStruct((O, N), jnp.float32),
    )(Y1t, S, CNT, b2, pw2)


def kernel(in_features, reduce_index, gather_index, W, b, prelu_w):
    X = in_features[0]
    C, N = X.shape
    O = W.shape[0]
    E = reduce_index.shape[0]
    ridx = reduce_index.astype(jnp.int32)
    gidx = gather_index.astype(jnp.int32)
    W1 = W[:, :C]
    W2 = W[:, C:]
    Wd_t = (W1 - W2).T
    W2_t = W2.T

    Y1t, Y2t = _stage1(X, Wd_t, W2_t)

    z128 = jnp.zeros((CHUNK, O), jnp.float32)
    zhist = jnp.zeros((N,), jnp.float32)
    S, CNT = _make_sc_kernel(N, O, E)(Y2t, ridx, gidx, z128, zhist)

    out2d = _stage3(Y1t, S, CNT.reshape(NC * NS, N),
                    b.reshape(1, O), prelu_w.reshape(1, 1))
    return out2d[None]


# speedup vs baseline: 3.2705x; 1.4248x over previous
import jax
import jax.numpy as jnp
from jax import lax
from jax.experimental import pallas as pl
from jax.experimental.pallas import tpu as pltpu
from jax.experimental.pallas import tpu_sc as plsc

NC = 2
NS = 16
NW = NC * NS
L = 16
CHUNK = 128


def _matmul_body(x_ref, wd_ref, w2_ref, y1_ref, y2_ref):
    x = x_ref[...]
    dn = (((0,), (0,)), ((), ()))
    y1_ref[...] = lax.dot_general(x, wd_ref[...], dn,
                                  preferred_element_type=jnp.float32)
    y2_ref[...] = lax.dot_general(x, w2_ref[...], dn,
                                  preferred_element_type=jnp.float32)


def _stage1(X, Wd_t, W2_t):
    C, N = X.shape
    O = Wd_t.shape[1]
    return pl.pallas_call(
        _matmul_body,
        out_shape=[
            jax.ShapeDtypeStruct((N, O), jnp.float32),
            jax.ShapeDtypeStruct((N, O), jnp.float32),
        ],
    )(X, Wd_t, W2_t)


def _make_sc_kernel(N, O, E):
    assert E % CHUNK == 0
    tot_chunks = E // CHUNK
    base_chunks = tot_chunks // NW
    extra = tot_chunks % NW
    row_grps = N // CHUNK
    row_tail = N - row_grps * CHUNK
    assert row_tail % 8 == 0

    mesh = plsc.VectorSubcoreMesh(core_axis_name="c", subcore_axis_name="s")

    assert base_chunks % 2 == 0

    def body(y2_hbm, ridx_hbm, gidx_hbm, z128_hbm, zhist_hbm,
             s_out, cnt_out,
             s_sh, idxg0, idxg1, idxr0, idxr1, rows0, rows1,
             hist_v, sem0, sem1):
        cid = lax.axis_index("c")
        sid = lax.axis_index("s")
        wid = sid * NC + cid
        idxg = (idxg0, idxg1)
        idxr = (idxr0, idxr1)
        rows = (rows0, rows1)
        sems = (sem0, sem1)
        rows_v = rows0

        pltpu.sync_copy(z128_hbm, rows_v)
        pltpu.sync_copy(zhist_hbm, hist_v)

        def zero_grp(g, carry):
            o = (sid + g * NS) * CHUNK
            pltpu.sync_copy(rows_v, s_sh.at[pl.ds(o, CHUNK)])
            return carry

        lax.fori_loop(0, row_grps // NS, zero_grp, 0)
        rem = row_grps % NS

        @pl.when(sid < rem)
        def _zero_rem():
            o = ((row_grps // NS) * NS + sid) * CHUNK
            pltpu.sync_copy(rows_v, s_sh.at[pl.ds(o, CHUNK)])

        if row_tail:
            @pl.when(sid == NS - 1)
            def _zero_tail():
                t0 = row_grps * CHUNK
                pltpu.sync_copy(rows_v.at[pl.ds(0, row_tail)],
                                s_sh.at[pl.ds(t0, row_tail)])
        plsc.subcore_barrier()

        def load_idx(j, b):
            base = (wid * base_chunks + j) * CHUNK
            pltpu.sync_copy(gidx_hbm.at[pl.ds(base, CHUNK)], idxg[b])
            pltpu.sync_copy(ridx_hbm.at[pl.ds(base, CHUNK)], idxr[b])

        def start_gather(b):
            pltpu.make_async_copy(y2_hbm.at[idxg[b]], rows[b],
                                  sems[b]).start()

        def hist(b):
            for u in range(CHUNK // L):
                iv = idxr[b][pl.ds(u * L, L)]
                cnts, last = plsc.scan_count(iv)
                plsc.addupdate_scatter(hist_v, [iv],
                                       cnts.astype(jnp.float32), mask=last)

        def wait_gather(b):
            pltpu.make_async_copy(y2_hbm.at[idxg[b]], rows[b],
                                  sems[b]).wait()

        def scatter(b):
            pltpu.sync_copy(rows[b], s_sh.at[idxr[b]], add=True)

        load_idx(0, 0)
        start_gather(0)

        def pair(t, carry):
            load_idx(2 * t + 1, 1)
            start_gather(1)
            hist(0)
            wait_gather(0)
            scatter(0)
            load_idx(2 * t + 2, 0)
            start_gather(0)
            hist(1)
            wait_gather(1)
            scatter(1)
            return carry

        lax.fori_loop(0, base_chunks // 2, pair, 0)
        wait_gather(0)
        if extra:
            @pl.when(wid < extra)
            def _extra():
                j = NW * base_chunks + wid
                base = j * CHUNK
                pltpu.sync_copy(gidx_hbm.at[pl.ds(base, CHUNK)], idxg0)
                pltpu.sync_copy(ridx_hbm.at[pl.ds(base, CHUNK)], idxr0)
                start_gather(0)
                hist(0)
                wait_gather(0)
                scatter(0)
        plsc.subcore_barrier()

        def pub_grp(g, carry):
            o = (sid + g * NS) * CHUNK
            pltpu.sync_copy(s_sh.at[pl.ds(o, CHUNK)], rows_v)
            pltpu.sync_copy(rows_v, s_out.at[cid, pl.ds(o, CHUNK)])
            return carry

        lax.fori_loop(0, row_grps // NS, pub_grp, 0)

        @pl.when(sid < rem)
        def _pub_rem():
            o = ((row_grps // NS) * NS + sid) * CHUNK
            pltpu.sync_copy(s_sh.at[pl.ds(o, CHUNK)], rows_v)
            pltpu.sync_copy(rows_v, s_out.at[cid, pl.ds(o, CHUNK)])

        if row_tail:
            @pl.when(sid == NS - 1)
            def _pub_tail():
                t0 = row_grps * CHUNK
                pltpu.sync_copy(s_sh.at[pl.ds(t0, row_tail)],
                                rows_v.at[pl.ds(0, row_tail)])
                pltpu.sync_copy(rows_v.at[pl.ds(0, row_tail)],
                                s_out.at[cid, pl.ds(t0, row_tail)])

        pltpu.sync_copy(hist_v, cnt_out.at[cid, sid])

    return pl.kernel(
        body,
        out_type=[
            jax.ShapeDtypeStruct((NC, N, O), jnp.float32),
            jax.ShapeDtypeStruct((NC, NS, N), jnp.float32),
        ],
        mesh=mesh,
        compiler_params=pltpu.CompilerParams(needs_layout_passes=False),
        scratch_types=[
            pltpu.VMEM_SHARED((N, O), jnp.float32),
            pltpu.VMEM((CHUNK,), jnp.int32),
            pltpu.VMEM((CHUNK,), jnp.int32),
            pltpu.VMEM((CHUNK,), jnp.int32),
            pltpu.VMEM((CHUNK,), jnp.int32),
            pltpu.VMEM((CHUNK, O), jnp.float32),
            pltpu.VMEM((CHUNK, O), jnp.float32),
            pltpu.VMEM((N,), jnp.float32),
            pltpu.SemaphoreType.DMA,
            pltpu.SemaphoreType.DMA,
        ],
    )


def _combine_body(y1_ref, s_ref, cnt_ref, b_ref, pw_ref, out_ref):
    s = s_ref[0] + s_ref[1]
    c = jnp.sum(cnt_ref[...], axis=0)[:, None]
    y = y1_ref[...] + b_ref[...]
    tot = c * y + s
    out = tot / jnp.maximum(c, 1.0)
    pw = pw_ref[0, 0]
    out = jnp.where(out >= 0, out, pw * out)
    out_ref[...] = out.T


def _stage3(Y1t, S, CNT, b2, pw2):
    N, O = Y1t.shape
    return pl.pallas_call(
        _combine_body,
        out_shape=jax.ShapeDtypeStruct((O, N), jnp.float32),
    )(Y1t, S, CNT, b2, pw2)


def kernel(in_features, reduce_index, gather_index, W, b, prelu_w):
    X = in_features[0]
    C, N = X.shape
    O = W.shape[0]
    E = reduce_index.shape[0]
    ridx = reduce_index.astype(jnp.int32)
    gidx = gather_index.astype(jnp.int32)
    W1 = W[:, :C]
    W2 = W[:, C:]
    Wd_t = (W1 - W2).T
    W2_t = W2.T

    Y1t, Y2t = _stage1(X, Wd_t, W2_t)

    z128 = jnp.zeros((CHUNK, O), jnp.float32)
    zhist = jnp.zeros((N,), jnp.float32)
    S, CNT = _make_sc_kernel(N, O, E)(Y2t, ridx, gidx, z128, zhist)

    out2d = _stage3(Y1t, S, CNT.reshape(NC * NS, N),
                    b.reshape(1, O), prelu_w.reshape(1, 1))
    return out2d[None]


# speedup vs baseline: 3.5600x; 1.0885x over previous
import jax
import jax.numpy as jnp
from jax import lax
from jax.experimental import pallas as pl
from jax.experimental.pallas import tpu as pltpu
from jax.experimental.pallas import tpu_sc as plsc

NC = 2
NS = 16
NW = NC * NS
L = 16
CHUNK = 128


def _matmul_body(x_ref, wd_ref, w2_ref, y1_ref, y2_ref):
    x = x_ref[...]
    dn = (((0,), (0,)), ((), ()))
    y1_ref[...] = lax.dot_general(x, wd_ref[...], dn,
                                  preferred_element_type=jnp.float32)
    y2_ref[...] = lax.dot_general(x, w2_ref[...], dn,
                                  preferred_element_type=jnp.float32)


def _stage1(X, Wd_t, W2_t):
    C, N = X.shape
    O = Wd_t.shape[1]
    return pl.pallas_call(
        _matmul_body,
        out_shape=[
            jax.ShapeDtypeStruct((N, O), jnp.float32),
            jax.ShapeDtypeStruct((N, O), jnp.float32),
        ],
    )(X, Wd_t, W2_t)


def _make_sc_kernel(N, O, E):
    assert E % CHUNK == 0
    tot_chunks = E // CHUNK
    base_chunks = tot_chunks // NW
    extra = tot_chunks % NW
    row_grps = N // CHUNK
    row_tail = N - row_grps * CHUNK
    assert row_tail % 8 == 0

    mesh = plsc.VectorSubcoreMesh(core_axis_name="c", subcore_axis_name="s")

    assert base_chunks % 2 == 0

    def body(y2_hbm, ridx_hbm, gidx_hbm, z128_hbm, zhist_hbm,
             s_out, cnt_out,
             s_sh, idxg0, idxg1, idxr0, idxr1, rows0, rows1,
             hist_v, sem0, sem1, semi0, semi1):
        cid = lax.axis_index("c")
        sid = lax.axis_index("s")
        wid = sid * NC + cid
        idxg = (idxg0, idxg1)
        idxr = (idxr0, idxr1)
        rows = (rows0, rows1)
        sems = (sem0, sem1)
        semi = (semi0, semi1)
        rows_v = rows0

        pltpu.sync_copy(z128_hbm, rows_v)
        pltpu.sync_copy(zhist_hbm, hist_v)

        def zero_grp(g, carry):
            o = (sid + g * NS) * CHUNK
            pltpu.sync_copy(rows_v, s_sh.at[pl.ds(o, CHUNK)])
            return carry

        lax.fori_loop(0, row_grps // NS, zero_grp, 0)
        rem = row_grps % NS

        @pl.when(sid < rem)
        def _zero_rem():
            o = ((row_grps // NS) * NS + sid) * CHUNK
            pltpu.sync_copy(rows_v, s_sh.at[pl.ds(o, CHUNK)])

        if row_tail:
            @pl.when(sid == NS - 1)
            def _zero_tail():
                t0 = row_grps * CHUNK
                pltpu.sync_copy(rows_v.at[pl.ds(0, row_tail)],
                                s_sh.at[pl.ds(t0, row_tail)])
        plsc.subcore_barrier()

        def load_idx(j, b):
            base = (wid * base_chunks + j) * CHUNK
            pltpu.sync_copy(gidx_hbm.at[pl.ds(base, CHUNK)], idxg[b])
            pltpu.sync_copy(ridx_hbm.at[pl.ds(base, CHUNK)], idxr[b])

        def prefetch_idx(j, b):
            base = (wid * base_chunks + j) * CHUNK
            pltpu.make_async_copy(gidx_hbm.at[pl.ds(base, CHUNK)],
                                  idxg[b], semi[b]).start()
            pltpu.make_async_copy(ridx_hbm.at[pl.ds(base, CHUNK)],
                                  idxr[b], semi[b]).start()

        def wait_idx(b):
            pltpu.make_async_copy(gidx_hbm.at[pl.ds(0, CHUNK)],
                                  idxg[b], semi[b]).wait()
            pltpu.make_async_copy(ridx_hbm.at[pl.ds(0, CHUNK)],
                                  idxr[b], semi[b]).wait()

        def start_gather(b):
            pltpu.make_async_copy(y2_hbm.at[idxg[b]], rows[b],
                                  sems[b]).start()

        def hist(b):
            for u in range(CHUNK // L):
                iv = idxr[b][pl.ds(u * L, L)]
                cnts, last = plsc.scan_count(iv)
                plsc.addupdate_scatter(hist_v, [iv],
                                       cnts.astype(jnp.float32), mask=last)

        def wait_gather(b):
            pltpu.make_async_copy(y2_hbm.at[idxg[b]], rows[b],
                                  sems[b]).wait()

        def scatter(b):
            pltpu.sync_copy(rows[b], s_sh.at[idxr[b]], add=True)

        load_idx(0, 0)
        start_gather(0)
        prefetch_idx(1, 1)

        def pair(t, carry):
            hist(0)
            wait_idx(1)
            start_gather(1)
            wait_gather(0)
            scatter(0)
            prefetch_idx(2 * t + 2, 0)
            hist(1)
            wait_gather(1)
            scatter(1)
            wait_idx(0)
            start_gather(0)
            prefetch_idx(2 * t + 3, 1)
            return carry

        lax.fori_loop(0, base_chunks // 2, pair, 0)
        wait_gather(0)
        wait_idx(1)
        if extra:
            @pl.when(wid < extra)
            def _extra():
                j = NW * base_chunks + wid
                base = j * CHUNK
                pltpu.sync_copy(gidx_hbm.at[pl.ds(base, CHUNK)], idxg0)
                pltpu.sync_copy(ridx_hbm.at[pl.ds(base, CHUNK)], idxr0)
                start_gather(0)
                hist(0)
                wait_gather(0)
                scatter(0)
        plsc.subcore_barrier()

        def pub_grp(g, carry):
            o = (sid + g * NS) * CHUNK
            pltpu.sync_copy(s_sh.at[pl.ds(o, CHUNK)], rows_v)
            pltpu.sync_copy(rows_v, s_out.at[cid, pl.ds(o, CHUNK)])
            return carry

        lax.fori_loop(0, row_grps // NS, pub_grp, 0)

        @pl.when(sid < rem)
        def _pub_rem():
            o = ((row_grps // NS) * NS + sid) * CHUNK
            pltpu.sync_copy(s_sh.at[pl.ds(o, CHUNK)], rows_v)
            pltpu.sync_copy(rows_v, s_out.at[cid, pl.ds(o, CHUNK)])

        if row_tail:
            @pl.when(sid == NS - 1)
            def _pub_tail():
                t0 = row_grps * CHUNK
                pltpu.sync_copy(s_sh.at[pl.ds(t0, row_tail)],
                                rows_v.at[pl.ds(0, row_tail)])
                pltpu.sync_copy(rows_v.at[pl.ds(0, row_tail)],
                                s_out.at[cid, pl.ds(t0, row_tail)])

        pltpu.sync_copy(hist_v, cnt_out.at[cid, sid])

    return pl.kernel(
        body,
        out_type=[
            jax.ShapeDtypeStruct((NC, N, O), jnp.float32),
            jax.ShapeDtypeStruct((NC, NS, N), jnp.float32),
        ],
        mesh=mesh,
        compiler_params=pltpu.CompilerParams(needs_layout_passes=False),
        scratch_types=[
            pltpu.VMEM_SHARED((N, O), jnp.float32),
            pltpu.VMEM((CHUNK,), jnp.int32),
            pltpu.VMEM((CHUNK,), jnp.int32),
            pltpu.VMEM((CHUNK,), jnp.int32),
            pltpu.VMEM((CHUNK,), jnp.int32),
            pltpu.VMEM((CHUNK, O), jnp.float32),
            pltpu.VMEM((CHUNK, O), jnp.float32),
            pltpu.VMEM((N,), jnp.float32),
            pltpu.SemaphoreType.DMA,
            pltpu.SemaphoreType.DMA,
            pltpu.SemaphoreType.DMA,
            pltpu.SemaphoreType.DMA,
        ],
    )


def _combine_body(y1_ref, s_ref, cnt_ref, b_ref, pw_ref, out_ref):
    s = s_ref[0] + s_ref[1]
    c = jnp.sum(cnt_ref[...], axis=0)[:, None]
    y = y1_ref[...] + b_ref[...]
    tot = c * y + s
    out = tot / jnp.maximum(c, 1.0)
    pw = pw_ref[0, 0]
    out = jnp.where(out >= 0, out, pw * out)
    out_ref[...] = out.T


def _stage3(Y1t, S, CNT, b2, pw2):
    N, O = Y1t.shape
    return pl.pallas_call(
        _combine_body,
        out_shape=jax.ShapeDtypeStruct((O, N), jnp.float32),
    )(Y1t, S, CNT, b2, pw2)


def kernel(in_features, reduce_index, gather_index, W, b, prelu_w):
    X = in_features[0]
    C, N = X.shape
    O = W.shape[0]
    E = reduce_index.shape[0]
    ridx = reduce_index.astype(jnp.int32)
    gidx = gather_index.astype(jnp.int32)
    W1 = W[:, :C]
    W2 = W[:, C:]
    Wd_t = (W1 - W2).T
    W2_t = W2.T

    Y1t, Y2t = _stage1(X, Wd_t, W2_t)

    z128 = jnp.zeros((CHUNK, O), jnp.float32)
    zhist = jnp.zeros((N,), jnp.float32)
    S, CNT = _make_sc_kernel(N, O, E)(Y2t, ridx, gidx, z128, zhist)

    out2d = _stage3(Y1t, S, CNT.reshape(NC * NS, N),
                    b.reshape(1, O), prelu_w.reshape(1, 1))
    return out2d[None]
